# baseline (speedup 1.0000x reference)
import functools

import jax
import jax.numpy as jnp
from jax import lax
from jax.experimental import pallas as pl
from jax.experimental.pallas import tpu as pltpu
from jax.experimental.pallas import tpu_sc as plsc

NUM_CORES = 2
NUM_SUBCORES = 16
NUM_WORKERS = NUM_CORES * NUM_SUBCORES
LANES = 16
CHUNK = 256
SUB = 8


def _sc_lookup_concat(flat_tables, i2p, i3p, i4p, vocab):
    _, dim = flat_tables.shape
    seq, bblocks, heads, lane = i2p.shape
    batch = bblocks * lane
    tokens = batch * seq
    num_tables = 3 * heads
    width = num_tables * dim
    per_w = tokens // NUM_WORKERS
    n_chunks = per_w // CHUNK
    t8 = CHUNK // SUB
    mesh = plsc.VectorSubcoreMesh(core_axis_name="c", subcore_axis_name="s")

    @functools.partial(
        pl.kernel,
        mesh=mesh,
        out_type=jax.ShapeDtypeStruct(
            (tokens // SUB, width // 128, SUB, 128), jnp.float32),
        compiler_params=pltpu.CompilerParams(
            use_tc_tiling_on_sc=False, needs_layout_passes=False),
        scratch_types=[
            pltpu.VMEM((3, CHUNK // 128, heads, lane), jnp.int32),
            pltpu.VMEM((2, num_tables, CHUNK), jnp.int32),
            pltpu.VMEM((2, CHUNK, dim), jnp.float32),
            pltpu.SemaphoreType.DMA,
            pltpu.SemaphoreType.DMA,
            pltpu.SemaphoreType.DMA,
            pltpu.SemaphoreType.DMA,
        ],
    )
    def k(tab_hbm, i2_hbm, i3_hbm, i4_hbm, out_hbm, blk_v, idx_v, rows_v,
          g0, g1, w0, w1):
        gsem = (g0, g1)
        wsem = (w0, w1)
        idx_srcs = (i2_hbm, i3_hbm, i4_hbm)
        wid = lax.axis_index("s") * NUM_CORES + lax.axis_index("c")
        tb0 = wid * (per_w // SUB)

        def phase_a(c, cb):
            t8a = tb0 + c * t8
            s = t8a // (batch // SUB)
            b0 = t8a % (batch // SUB) // (lane // SUB)
            for n in range(3):
                pltpu.sync_copy(idx_srcs[n].at[s, pl.ds(b0, CHUNK // 128)],
                                blk_v.at[n])
            for g in range(CHUNK // LANES):
                i0 = jnp.full((LANES,), g % 2, jnp.int32)
                il = lax.iota(jnp.int32, LANES) * SUB + g // 2
                for n in range(3):
                    for h in range(heads):
                        t = n * heads + h
                        v = plsc.load_gather(
                            blk_v.at[n],
                            [i0, jnp.full((LANES,), h, jnp.int32), il])
                        idx_v[cb, t, pl.ds(g * LANES, LANES)] = v

        def start_gather(t, b, cb):
            pltpu.async_copy(tab_hbm.at[idx_v.at[cb, t]], rows_v.at[b], gsem[b])

        def wait_gather(b):
            pltpu.make_async_copy(
                tab_hbm.at[idx_v.at[0, 0]], rows_v.at[b], gsem[b]).wait()

        def start_write(c, t, b):
            blk = tb0 + c * t8
            for b8 in range(SUB):
                pltpu.async_copy(
                    rows_v.at[b, pl.ds(b8 * t8, t8)],
                    out_hbm.at[pl.ds(blk, t8), t // 2, b8,
                               pl.ds((t % 2) * dim, dim)],
                    wsem[b])

        def wait_write(b):
            for _ in range(SUB):
                pltpu.make_async_copy(
                    rows_v.at[b, pl.ds(0, t8)],
                    out_hbm.at[pl.ds(0, t8), 0, 0, pl.ds(0, dim)],
                    wsem[b]).wait()

        phase_a(0, 0)
        start_gather(0, 0, 0)

        @pl.loop(0, (n_chunks - 1) // 2)
        def _(i):
            for cslot in range(2):
                cb = cslot
                c = 2 * i + cslot
                for t in range(num_tables):
                    b = t % 2

                    if cslot == 0 and t == 0:
                        @pl.when(i > 0)
                        def _():
                            wait_write(b ^ 1)
                    else:
                        wait_write(b ^ 1)

                    if t < num_tables - 1:
                        start_gather(t + 1, b ^ 1, cb)
                    else:
                        phase_a(c + 1, cb ^ 1)
                        start_gather(0, b ^ 1, cb ^ 1)

                    wait_gather(b)
                    start_write(c, t, b)

        c_last = n_chunks - 1
        for t in range(num_tables):
            b = t % 2
            wait_write(b ^ 1)
            if t < num_tables - 1:
                start_gather(t + 1, b ^ 1, 0)
            wait_gather(b)
            start_write(c_last, t, b)

        wait_write(1)

    return k(flat_tables, i2p, i3p, i4p)


def kernel(indices_2, indices_3, indices_4, tables):
    batch, seq, heads = indices_2.shape
    num_tables, vocab, dim = tables.shape
    width = num_tables * dim

    def as_tiles(ix, n):
        offs = (n * heads + jnp.arange(heads, dtype=jnp.int32)) * vocab
        return (ix.astype(jnp.int32).transpose(1, 2, 0)
                .reshape(seq, heads, batch // 128, 128).transpose(0, 2, 1, 3)
                + offs[None, None, :, None])

    out4 = _sc_lookup_concat(
        tables.reshape(num_tables * vocab, dim),
        as_tiles(indices_2, 0), as_tiles(indices_3, 1),
        as_tiles(indices_4, 2), vocab)
    out = (out4.reshape(seq, batch // SUB, width // 128, SUB, 128)
           .transpose(0, 1, 3, 2, 4).reshape(seq, batch, width)
           .transpose(1, 0, 2))
    return out


# speedup vs baseline: 1.0436x; 1.0436x over previous
import functools

import jax
import jax.numpy as jnp
from jax import lax
from jax.experimental import pallas as pl
from jax.experimental.pallas import tpu as pltpu
from jax.experimental.pallas import tpu_sc as plsc

NUM_CORES = 2
NUM_SUBCORES = 16
NUM_WORKERS = NUM_CORES * NUM_SUBCORES
LANES = 16
CHUNK = 256
SUB = 8


def _sc_lookup_concat(flat_tables, i2p, i3p, i4p, vocab):
    _, dim = flat_tables.shape
    seq, bblocks, heads, lane = i2p.shape
    batch = bblocks * lane
    tokens = batch * seq
    num_tables = 3 * heads
    width = num_tables * dim
    per_w = tokens // NUM_WORKERS
    n_chunks = per_w // CHUNK
    t8 = CHUNK // SUB
    mesh = plsc.VectorSubcoreMesh(core_axis_name="c", subcore_axis_name="s")

    @functools.partial(
        pl.kernel,
        mesh=mesh,
        out_type=jax.ShapeDtypeStruct(
            (tokens // SUB, width // 128, SUB, 128), jnp.float32),
        compiler_params=pltpu.CompilerParams(
            use_tc_tiling_on_sc=False, needs_layout_passes=False),
        scratch_types=[
            pltpu.VMEM((2, 3, CHUNK // 128, heads, lane), jnp.int32),
            pltpu.VMEM((2, num_tables, CHUNK), jnp.int32),
            pltpu.VMEM((2, CHUNK, dim), jnp.float32),
            pltpu.SemaphoreType.DMA,
            pltpu.SemaphoreType.DMA,
            pltpu.SemaphoreType.DMA,
            pltpu.SemaphoreType.DMA,
            pltpu.SemaphoreType.DMA,
        ],
    )
    def k(tab_hbm, i2_hbm, i3_hbm, i4_hbm, out_hbm, blk_v, idx_v, rows_v,
          g0, g1, w0, w1, bsem):
        gsem = (g0, g1)
        wsem = (w0, w1)
        idx_srcs = (i2_hbm, i3_hbm, i4_hbm)
        wid = lax.axis_index("s") * NUM_CORES + lax.axis_index("c")
        tb0 = wid * (per_w // SUB)

        def start_blk(c, cb):
            t8a = tb0 + c * t8
            s = t8a // (batch // SUB)
            b0 = t8a % (batch // SUB) // (lane // SUB)
            for n in range(3):
                pltpu.async_copy(idx_srcs[n].at[s, pl.ds(b0, CHUNK // 128)],
                                 blk_v.at[cb, n], bsem)

        def phase_a(c, cb):
            for n in range(3):
                pltpu.make_async_copy(
                    idx_srcs[n].at[0, pl.ds(0, CHUNK // 128)],
                    blk_v.at[cb, n], bsem).wait()
            for g in range(CHUNK // LANES):
                i0 = jnp.full((LANES,), g % 2, jnp.int32)
                il = lax.iota(jnp.int32, LANES) * SUB + g // 2
                for n in range(3):
                    for h in range(heads):
                        t = n * heads + h
                        v = plsc.load_gather(
                            blk_v.at[cb, n],
                            [i0, jnp.full((LANES,), h, jnp.int32), il])
                        idx_v[cb, t, pl.ds(g * LANES, LANES)] = v

        def start_gather(t, b, cb):
            pltpu.async_copy(tab_hbm.at[idx_v.at[cb, t]], rows_v.at[b], gsem[b])

        def wait_gather(b):
            pltpu.make_async_copy(
                tab_hbm.at[idx_v.at[0, 0]], rows_v.at[b], gsem[b]).wait()

        def start_write(c, t, b):
            blk = tb0 + c * t8
            for b8 in range(SUB):
                pltpu.async_copy(
                    rows_v.at[b, pl.ds(b8 * t8, t8)],
                    out_hbm.at[pl.ds(blk, t8), t // 2, b8,
                               pl.ds((t % 2) * dim, dim)],
                    wsem[b])

        def wait_write(b):
            for _ in range(SUB):
                pltpu.make_async_copy(
                    rows_v.at[b, pl.ds(0, t8)],
                    out_hbm.at[pl.ds(0, t8), 0, 0, pl.ds(0, dim)],
                    wsem[b]).wait()

        start_blk(0, 0)
        phase_a(0, 0)
        start_gather(0, 0, 0)

        @pl.loop(0, (n_chunks - 1) // 2)
        def _(i):
            for cslot in range(2):
                cb = cslot
                c = 2 * i + cslot
                for t in range(num_tables):
                    b = t % 2

                    if t == 0:
                        start_blk(c + 1, cb ^ 1)

                    if cslot == 0 and t == 0:
                        @pl.when(i > 0)
                        def _():
                            wait_write(b ^ 1)
                    else:
                        wait_write(b ^ 1)

                    if t < num_tables - 1:
                        start_gather(t + 1, b ^ 1, cb)
                    else:
                        phase_a(c + 1, cb ^ 1)
                        start_gather(0, b ^ 1, cb ^ 1)

                    wait_gather(b)
                    start_write(c, t, b)

        c_last = n_chunks - 1
        for t in range(num_tables):
            b = t % 2
            wait_write(b ^ 1)
            if t < num_tables - 1:
                start_gather(t + 1, b ^ 1, 0)
            wait_gather(b)
            start_write(c_last, t, b)

        wait_write(1)

    return k(flat_tables, i2p, i3p, i4p)


def kernel(indices_2, indices_3, indices_4, tables):
    batch, seq, heads = indices_2.shape
    num_tables, vocab, dim = tables.shape
    width = num_tables * dim

    def as_tiles(ix, n):
        offs = (n * heads + jnp.arange(heads, dtype=jnp.int32)) * vocab
        return (ix.astype(jnp.int32).transpose(1, 2, 0)
                .reshape(seq, heads, batch // 128, 128).transpose(0, 2, 1, 3)
                + offs[None, None, :, None])

    out4 = _sc_lookup_concat(
        tables.reshape(num_tables * vocab, dim),
        as_tiles(indices_2, 0), as_tiles(indices_3, 1),
        as_tiles(indices_4, 2), vocab)
    out = (out4.reshape(seq, batch // SUB, width // 128, SUB, 128)
           .transpose(0, 1, 3, 2, 4).reshape(seq, batch, width)
           .transpose(1, 0, 2))
    return out
